# baseline (device time: 14651 ns/iter reference)
import jax
import jax.numpy as jnp
from jax import lax
from jax.experimental import pallas as pl
from jax.experimental.pallas import tpu as pltpu

N_DEV = 8
B = 256
H = 8


def kernel(x):
    m, n = x.shape
    nblk = m // B
    assert m % B == 0 and nblk >= 2

    def body(x_hbm, out_hbm, xbuf, obuf, halo_ref, topbuf, rowfix,
             in_sems, out_sems, send_sems, recv_sems):
        my = lax.axis_index("i")
        is_first = my == 0
        is_last = my == N_DEV - 1

        barrier = pltpu.get_barrier_semaphore()

        @pl.when(jnp.logical_not(is_first))
        def _():
            pl.semaphore_signal(
                barrier, inc=1, device_id=(my - 1,),
                device_id_type=pl.DeviceIdType.MESH,
            )

        @pl.when(jnp.logical_not(is_last))
        def _():
            pl.semaphore_signal(
                barrier, inc=1, device_id=(my + 1,),
                device_id_type=pl.DeviceIdType.MESH,
            )

        @pl.when(is_first | is_last)
        def _():
            pl.semaphore_wait(barrier, 1)

        @pl.when(jnp.logical_not(is_first | is_last))
        def _():
            pl.semaphore_wait(barrier, 2)

        send_right = pltpu.make_async_remote_copy(
            src_ref=x_hbm.at[pl.ds(m - 1, 1), :],
            dst_ref=halo_ref.at[0],
            send_sem=send_sems.at[0],
            recv_sem=recv_sems.at[0],
            device_id=(my + 1,),
            device_id_type=pl.DeviceIdType.MESH,
        )
        send_left = pltpu.make_async_remote_copy(
            src_ref=x_hbm.at[pl.ds(0, 1), :],
            dst_ref=halo_ref.at[1],
            send_sem=send_sems.at[1],
            recv_sem=recv_sems.at[1],
            device_id=(my - 1,),
            device_id_type=pl.DeviceIdType.MESH,
        )

        @pl.when(jnp.logical_not(is_last))
        def _():
            send_right.start()

        @pl.when(jnp.logical_not(is_first))
        def _():
            send_left.start()

        def copy_in(k, slot):
            if k == 0:
                return pltpu.make_async_copy(
                    x_hbm.at[pl.ds(0, B + H), :],
                    xbuf.at[slot, pl.ds(H, B + H), :],
                    in_sems.at[slot],
                )
            if k == nblk - 1:
                return pltpu.make_async_copy(
                    x_hbm.at[pl.ds(k * B - H, B + H), :],
                    xbuf.at[slot, pl.ds(0, B + H), :],
                    in_sems.at[slot],
                )
            return pltpu.make_async_copy(
                x_hbm.at[pl.ds(k * B - H, B + 2 * H), :],
                xbuf.at[slot, pl.ds(0, B + 2 * H), :],
                in_sems.at[slot],
            )

        def copy_out(k, slot):
            return pltpu.make_async_copy(
                obuf.at[slot],
                out_hbm.at[pl.ds(k * B, B), :],
                out_sems.at[slot],
            )

        copy_in(0, 0).start()
        for k in range(nblk):
            slot = k % 2
            if k + 1 < nblk:
                copy_in(k + 1, (k + 1) % 2).start()
            copy_in(k, slot).wait()
            if k == 0:
                topbuf[0, :, :] = xbuf[0, pl.ds(H, 2), :]
            if k >= 2:
                copy_out(k - 2, slot).wait()
            obuf[slot, :, :] = (
                0.25 * xbuf[slot, pl.ds(H - 1, B), :]
                + 0.5 * xbuf[slot, pl.ds(H, B), :]
                + 0.25 * xbuf[slot, pl.ds(H + 1, B), :]
            )
            copy_out(k, slot).start()
        copy_out(nblk - 2, nblk % 2).wait()
        copy_out(nblk - 1, (nblk - 1) % 2).wait()

        last_slot = (nblk - 1) % 2

        @pl.when(is_first)
        def _():
            rowfix[0, :, :] = topbuf[0, pl.ds(0, 1), :]

        @pl.when(jnp.logical_not(is_first))
        def _():
            send_right.wait_recv()
            rowfix[0, :, :] = (
                0.25 * halo_ref[0]
                + 0.5 * topbuf[0, pl.ds(0, 1), :]
                + 0.25 * topbuf[0, pl.ds(1, 1), :]
            )

        @pl.when(is_last)
        def _():
            rowfix[1, :, :] = xbuf[last_slot, pl.ds(B + H - 1, 1), :]

        @pl.when(jnp.logical_not(is_last))
        def _():
            send_left.wait_recv()
            rowfix[1, :, :] = (
                0.25 * xbuf[last_slot, pl.ds(B + H - 2, 1), :]
                + 0.5 * xbuf[last_slot, pl.ds(B + H - 1, 1), :]
                + 0.25 * halo_ref[1]
            )

        fix_top = pltpu.make_async_copy(
            rowfix.at[0], out_hbm.at[pl.ds(0, 1), :], out_sems.at[0],
        )
        fix_bot = pltpu.make_async_copy(
            rowfix.at[1], out_hbm.at[pl.ds(m - 1, 1), :], out_sems.at[1],
        )
        fix_top.start()
        fix_bot.start()
        fix_top.wait()
        fix_bot.wait()

        @pl.when(jnp.logical_not(is_last))
        def _():
            send_right.wait_send()

        @pl.when(jnp.logical_not(is_first))
        def _():
            send_left.wait_send()

    return pl.pallas_call(
        body,
        out_shape=jax.ShapeDtypeStruct((m, n), x.dtype),
        in_specs=[pl.BlockSpec(memory_space=pl.ANY)],
        out_specs=pl.BlockSpec(memory_space=pl.ANY),
        scratch_shapes=[
            pltpu.VMEM((2, B + 2 * H, n), x.dtype),
            pltpu.VMEM((2, B, n), x.dtype),
            pltpu.VMEM((2, 1, n), x.dtype),
            pltpu.VMEM((1, 2, n), x.dtype),
            pltpu.VMEM((2, 1, n), x.dtype),
            pltpu.SemaphoreType.DMA((2,)),
            pltpu.SemaphoreType.DMA((2,)),
            pltpu.SemaphoreType.DMA((2,)),
            pltpu.SemaphoreType.DMA((2,)),
        ],
        compiler_params=pltpu.CompilerParams(collective_id=0),
    )(x)
